# Initial kernel scaffold; baseline (speedup 1.0000x reference)
import jax
import jax.numpy as jnp
from jax.experimental import pallas as pl


def kernel(x, boxes):
    raise NotImplementedError("write your pallas kernel here")


# speedup vs baseline: 54.4653x; 54.4653x over previous
import functools

import jax
import jax.numpy as jnp
import numpy as np
from jax import lax
from jax.experimental import pallas as pl
from jax.experimental.pallas import tpu as pltpu

_I32_MIN = np.int32(-(2**31))
_I32_MAXMAG = np.int32(0x7FFFFFFF)


def _heatmap_kernel(boxes_ref, x_ref, loss_ref, label_ref,
                    lab_scr, key_scr, arr_scr, *, W, H, NB, NIMG):
    i = pl.program_id(0)
    x = x_ref[0]

    rows = lax.broadcasted_iota(jnp.int32, (W, 1), 0)
    cols = lax.broadcasted_iota(jnp.int32, (1, H), 1)
    mask = jnp.zeros((W, H), dtype=jnp.bool_)
    area = jnp.int32(0)
    for j in range(NB):
        x1 = boxes_ref[i, j, 0] // 8
        y1 = boxes_ref[i, j, 1] // 8
        x2 = boxes_ref[i, j, 2] // 8
        y2 = boxes_ref[i, j, 3] // 8
        rin = (rows >= y1) & (rows < y2)
        cin = (cols >= x1) & (cols < x2)
        mask = mask | (rin & cin)
        area = area + (x2 - x1) * (y2 - y1)

    lab = jnp.where(mask, jnp.float32(0.0), x)
    lab_scr[...] = lab

    bits = lax.bitcast_convert_type(lab, jnp.int32)
    key = jnp.where(bits >= 0, bits, bits ^ _I32_MAXMAG)
    key_scr[...] = key

    K = area

    def _count_ge(c):
        return jnp.sum((key_scr[...] >= c).astype(jnp.int32))

    T0 = jnp.where(_count_ge(jnp.int32(0)) >= K, jnp.int32(0), _I32_MIN)

    def _key_body(it, T):
        b = 30 - it
        cand = T | lax.shift_left(jnp.int32(1), b)
        return jnp.where(_count_ge(cand) >= K, cand, T)

    kstar = lax.fori_loop(0, 31, _key_body, T0)

    c_gt = jnp.sum((key > kstar).astype(jnp.int32))
    t = K - c_gt

    idx = rows * H + cols
    arr_scr[...] = jnp.where(key == kstar, idx, _I32_MAXMAG)

    def _idx_body(it, p):
        b = 17 - it
        cand = p | lax.shift_left(jnp.int32(1), b)
        c = jnp.sum((arr_scr[...] < cand).astype(jnp.int32))
        return jnp.where(c < t, cand, p)

    idxstar = lax.fori_loop(0, 18, _idx_body, jnp.int32(0))

    selected = (key > kstar) | (arr_scr[...] <= idxstar)
    label = jnp.where(selected, jnp.float32(1.0), lab_scr[...])
    label_ref[0] = label

    @pl.when(i == 0)
    def _():
        loss_ref[...] = jnp.zeros((1, 1), jnp.float32)

    loss_ref[...] += jnp.sum(jnp.abs(x - label), keepdims=True)


def kernel(x, boxes):
    B, W, H = x.shape
    NB = boxes.shape[1]

    grid_spec = pltpu.PrefetchScalarGridSpec(
        num_scalar_prefetch=1,
        grid=(B,),
        in_specs=[pl.BlockSpec((1, W, H), lambda i, b: (i, 0, 0))],
        out_specs=[
            pl.BlockSpec((1, 1), lambda i, b: (0, 0)),
            pl.BlockSpec((1, W, H), lambda i, b: (i, 0, 0)),
        ],
        scratch_shapes=[
            pltpu.VMEM((W, H), jnp.float32),
            pltpu.VMEM((W, H), jnp.int32),
            pltpu.VMEM((W, H), jnp.int32),
        ],
    )
    loss_sum, label = pl.pallas_call(
        functools.partial(_heatmap_kernel, W=W, H=H, NB=NB, NIMG=B),
        grid_spec=grid_spec,
        out_shape=[
            jax.ShapeDtypeStruct((1, 1), jnp.float32),
            jax.ShapeDtypeStruct((B, W, H), jnp.float32),
        ],
    )(boxes, x)
    loss = loss_sum[0, 0] / jnp.float32(B * W * H)
    return (loss, x, label)


# speedup vs baseline: 85.7393x; 1.5742x over previous
import functools

import jax
import jax.numpy as jnp
import numpy as np
from jax import lax
from jax.experimental import pallas as pl
from jax.experimental.pallas import tpu as pltpu

_I32_MIN = np.int32(-(2**31))
_I32_MAXMAG = np.int32(0x7FFFFFFF)


def _keys_kernel(boxes_ref, x_ref, key_ref, *, W, H, NB):
    i = pl.program_id(0)
    x = x_ref[0]

    rows = lax.broadcasted_iota(jnp.int32, (W, 1), 0)
    cols = lax.broadcasted_iota(jnp.int32, (1, H), 1)
    mask = jnp.zeros((W, H), dtype=jnp.bool_)
    for j in range(NB):
        x1 = boxes_ref[i, j, 0] // 8
        y1 = boxes_ref[i, j, 1] // 8
        x2 = boxes_ref[i, j, 2] // 8
        y2 = boxes_ref[i, j, 3] // 8
        rin = (rows >= y1) & (rows < y2)
        cin = (cols >= x1) & (cols < x2)
        mask = mask | (rin & cin)

    lab = jnp.where(mask, jnp.float32(0.0), x)
    bits = lax.bitcast_convert_type(lab, jnp.int32)
    key_ref[0] = jnp.where(bits >= 0, bits, bits ^ _I32_MAXMAG)


def _select_kernel(keys_ref, boxes_ref, kstar_ref, idxstar_ref, arr_scr,
                   *, B, N, NB):
    keys = keys_ref[...]

    b = boxes_ref[...] // 8
    wdt = b[:, :, 2] - b[:, :, 0]
    hgt = b[:, :, 3] - b[:, :, 1]
    K = jnp.sum(wdt * hgt, axis=1, keepdims=True)

    def _count_ge(cand):
        return jnp.sum((keys >= cand).astype(jnp.int32), axis=1, keepdims=True)

    zero = jnp.zeros((B, 1), jnp.int32)
    T0 = jnp.where(_count_ge(zero) >= K, zero, jnp.full((B, 1), _I32_MIN))

    def _key_body(it, T):
        bit = lax.shift_left(jnp.int32(1), 30 - it)
        cand = T | bit
        return jnp.where(_count_ge(cand) >= K, cand, T)

    kstar = lax.fori_loop(0, 31, _key_body, T0)

    c_gt = jnp.sum((keys > kstar).astype(jnp.int32), axis=1, keepdims=True)
    t = K - c_gt

    lane = lax.broadcasted_iota(jnp.int32, (B, N), 1)
    arr_scr[...] = jnp.where(keys == kstar, lane, _I32_MAXMAG)

    def _idx_body(it, p):
        bit = lax.shift_left(jnp.int32(1), 17 - it)
        cand = p | bit
        c = jnp.sum((arr_scr[...] < cand).astype(jnp.int32), axis=1,
                    keepdims=True)
        return jnp.where(c < t, cand, p)

    idxstar = lax.fori_loop(0, 18, _idx_body, jnp.zeros((B, 1), jnp.int32))

    kstar_ref[...] = kstar
    idxstar_ref[...] = idxstar


def _final_kernel(kstar_ref, idxstar_ref, x_ref, key_ref, loss_ref, label_ref,
                  *, W, H):
    i = pl.program_id(0)
    x = x_ref[0]
    key = key_ref[0]
    ks = kstar_ref[i]
    istar = idxstar_ref[i]

    rows = lax.broadcasted_iota(jnp.int32, (W, 1), 0)
    cols = lax.broadcasted_iota(jnp.int32, (1, H), 1)
    idx = rows * H + cols

    sel = (key > ks) | ((key == ks) & (idx <= istar))
    lab = lax.bitcast_convert_type(
        jnp.where(key >= 0, key, key ^ _I32_MAXMAG), jnp.float32)
    label = jnp.where(sel, jnp.float32(1.0), lab)
    label_ref[0] = label

    @pl.when(i == 0)
    def _():
        loss_ref[...] = jnp.zeros((1, 1), jnp.float32)

    loss_ref[...] += jnp.sum(jnp.abs(x - label), keepdims=True)


def kernel(x, boxes):
    B, W, H = x.shape
    NB = boxes.shape[1]
    N = W * H

    keys = pl.pallas_call(
        functools.partial(_keys_kernel, W=W, H=H, NB=NB),
        grid_spec=pltpu.PrefetchScalarGridSpec(
            num_scalar_prefetch=1,
            grid=(B,),
            in_specs=[pl.BlockSpec((1, W, H), lambda i, b: (i, 0, 0))],
            out_specs=pl.BlockSpec((1, W, H), lambda i, b: (i, 0, 0)),
        ),
        out_shape=jax.ShapeDtypeStruct((B, W, H), jnp.int32),
    )(boxes, x)

    keys2d = keys.reshape(B, N)
    kstar, idxstar = pl.pallas_call(
        functools.partial(_select_kernel, B=B, N=N, NB=NB),
        grid=(1,),
        in_specs=[
            pl.BlockSpec((B, N), lambda i: (0, 0)),
            pl.BlockSpec((B, NB, 4), lambda i: (0, 0, 0)),
        ],
        out_specs=[
            pl.BlockSpec((B, 1), lambda i: (0, 0)),
            pl.BlockSpec((B, 1), lambda i: (0, 0)),
        ],
        scratch_shapes=[pltpu.VMEM((B, N), jnp.int32)],
        out_shape=[
            jax.ShapeDtypeStruct((B, 1), jnp.int32),
            jax.ShapeDtypeStruct((B, 1), jnp.int32),
        ],
    )(keys2d, boxes)

    loss_sum, label = pl.pallas_call(
        functools.partial(_final_kernel, W=W, H=H),
        grid_spec=pltpu.PrefetchScalarGridSpec(
            num_scalar_prefetch=2,
            grid=(B,),
            in_specs=[
                pl.BlockSpec((1, W, H), lambda i, a, c: (i, 0, 0)),
                pl.BlockSpec((1, W, H), lambda i, a, c: (i, 0, 0)),
            ],
            out_specs=[
                pl.BlockSpec((1, 1), lambda i, a, c: (0, 0)),
                pl.BlockSpec((1, W, H), lambda i, a, c: (i, 0, 0)),
            ],
        ),
        out_shape=[
            jax.ShapeDtypeStruct((1, 1), jnp.float32),
            jax.ShapeDtypeStruct((B, W, H), jnp.float32),
        ],
    )(kstar.reshape(B), idxstar.reshape(B), x, keys)

    loss = loss_sum[0, 0] / jnp.float32(B * W * H)
    return (loss, x, label)


# speedup vs baseline: 109.1778x; 1.2734x over previous
import functools

import jax
import jax.numpy as jnp
import numpy as np
from jax import lax
from jax.experimental import pallas as pl
from jax.experimental.pallas import tpu as pltpu

_I32_MIN = np.int32(-(2**31))
_I32_MAXMAG = np.int32(0x7FFFFFFF)


def _keys_kernel(boxes_ref, x_ref, key_ref, *, W, H, NB):
    i = pl.program_id(0)
    x = x_ref[0]

    rows = lax.broadcasted_iota(jnp.int32, (W, 1), 0)
    cols = lax.broadcasted_iota(jnp.int32, (1, H), 1)
    mask = jnp.zeros((W, H), dtype=jnp.bool_)
    for j in range(NB):
        x1 = boxes_ref[i, j, 0] // 8
        y1 = boxes_ref[i, j, 1] // 8
        x2 = boxes_ref[i, j, 2] // 8
        y2 = boxes_ref[i, j, 3] // 8
        rin = (rows >= y1) & (rows < y2)
        cin = (cols >= x1) & (cols < x2)
        mask = mask | (rin & cin)

    lab = jnp.where(mask, jnp.float32(0.0), x)
    bits = lax.bitcast_convert_type(lab, jnp.int32)
    key_ref[0] = jnp.where(bits >= 0, bits, bits ^ _I32_MAXMAG)


def _select_kernel(keys_ref, boxes_ref, kstar_ref, idxstar_ref, arr_scr,
                   *, B, W, H, NB):
    keys = keys_ref[...]

    b = boxes_ref[...] // 8
    wdt = b[:, :, 2] - b[:, :, 0]
    hgt = b[:, :, 3] - b[:, :, 1]
    K = jnp.sum(wdt * hgt, axis=1, keepdims=True)[..., None]

    def _count_ge(cand):
        return jnp.sum((keys >= cand).astype(jnp.int32), axis=(1, 2),
                       keepdims=True)

    zero = jnp.zeros((B, 1, 1), jnp.int32)
    T0 = jnp.where(_count_ge(zero) >= K, zero, jnp.full((B, 1, 1), _I32_MIN))

    def _key_body(it, T):
        bit = lax.shift_left(jnp.int32(1), 30 - it)
        cand = T | bit
        return jnp.where(_count_ge(cand) >= K, cand, T)

    kstar = lax.fori_loop(0, 31, _key_body, T0)

    c_gt = jnp.sum((keys > kstar).astype(jnp.int32), axis=(1, 2),
                   keepdims=True)
    t = K - c_gt

    rows = lax.broadcasted_iota(jnp.int32, (1, W, H), 1)
    cols = lax.broadcasted_iota(jnp.int32, (1, W, H), 2)
    idx = rows * H + cols
    arr_scr[...] = jnp.where(keys == kstar, idx, _I32_MAXMAG)

    def _idx_body(it, p):
        bit = lax.shift_left(jnp.int32(1), 17 - it)
        cand = p | bit
        c = jnp.sum((arr_scr[...] < cand).astype(jnp.int32), axis=(1, 2),
                    keepdims=True)
        return jnp.where(c < t, cand, p)

    idxstar = lax.fori_loop(0, 18, _idx_body, jnp.zeros((B, 1, 1), jnp.int32))

    kstar_ref[...] = kstar
    idxstar_ref[...] = idxstar


def _final_kernel(kstar_ref, idxstar_ref, x_ref, key_ref, loss_ref, label_ref,
                  *, W, H):
    i = pl.program_id(0)
    x = x_ref[0]
    key = key_ref[0]
    ks = kstar_ref[i]
    istar = idxstar_ref[i]

    rows = lax.broadcasted_iota(jnp.int32, (W, 1), 0)
    cols = lax.broadcasted_iota(jnp.int32, (1, H), 1)
    idx = rows * H + cols

    sel = (key > ks) | ((key == ks) & (idx <= istar))
    lab = lax.bitcast_convert_type(
        jnp.where(key >= 0, key, key ^ _I32_MAXMAG), jnp.float32)
    label = jnp.where(sel, jnp.float32(1.0), lab)
    label_ref[0] = label

    @pl.when(i == 0)
    def _():
        loss_ref[...] = jnp.zeros((1, 1), jnp.float32)

    loss_ref[...] += jnp.sum(jnp.abs(x - label), keepdims=True)


def kernel(x, boxes):
    B, W, H = x.shape
    NB = boxes.shape[1]
    N = W * H

    keys = pl.pallas_call(
        functools.partial(_keys_kernel, W=W, H=H, NB=NB),
        grid_spec=pltpu.PrefetchScalarGridSpec(
            num_scalar_prefetch=1,
            grid=(B,),
            in_specs=[pl.BlockSpec((1, W, H), lambda i, b: (i, 0, 0))],
            out_specs=pl.BlockSpec((1, W, H), lambda i, b: (i, 0, 0)),
        ),
        out_shape=jax.ShapeDtypeStruct((B, W, H), jnp.int32),
    )(boxes, x)

    kstar, idxstar = pl.pallas_call(
        functools.partial(_select_kernel, B=B, W=W, H=H, NB=NB),
        grid=(1,),
        in_specs=[
            pl.BlockSpec((B, W, H), lambda i: (0, 0, 0)),
            pl.BlockSpec((B, NB, 4), lambda i: (0, 0, 0)),
        ],
        out_specs=[
            pl.BlockSpec((B, 1, 1), lambda i: (0, 0, 0)),
            pl.BlockSpec((B, 1, 1), lambda i: (0, 0, 0)),
        ],
        scratch_shapes=[pltpu.VMEM((B, W, H), jnp.int32)],
        out_shape=[
            jax.ShapeDtypeStruct((B, 1, 1), jnp.int32),
            jax.ShapeDtypeStruct((B, 1, 1), jnp.int32),
        ],
    )(keys, boxes)

    loss_sum, label = pl.pallas_call(
        functools.partial(_final_kernel, W=W, H=H),
        grid_spec=pltpu.PrefetchScalarGridSpec(
            num_scalar_prefetch=2,
            grid=(B,),
            in_specs=[
                pl.BlockSpec((1, W, H), lambda i, a, c: (i, 0, 0)),
                pl.BlockSpec((1, W, H), lambda i, a, c: (i, 0, 0)),
            ],
            out_specs=[
                pl.BlockSpec((1, 1), lambda i, a, c: (0, 0)),
                pl.BlockSpec((1, W, H), lambda i, a, c: (i, 0, 0)),
            ],
        ),
        out_shape=[
            jax.ShapeDtypeStruct((1, 1), jnp.float32),
            jax.ShapeDtypeStruct((B, W, H), jnp.float32),
        ],
    )(kstar.reshape(B), idxstar.reshape(B), x, keys)

    loss = loss_sum[0, 0] / jnp.float32(B * W * H)
    return (loss, x, label)


# speedup vs baseline: 132.8837x; 1.2171x over previous
import functools

import jax
import jax.numpy as jnp
import numpy as np
from jax import lax
from jax.experimental import pallas as pl
from jax.experimental.pallas import tpu as pltpu

_I32_MIN = np.int32(-(2**31))
_I32_MAXMAG = np.int32(0x7FFFFFFF)


def _keys_kernel(boxes_ref, x_ref, key_ref, *, W, H, NB):
    i = pl.program_id(0)
    x = x_ref[0]

    rows = lax.broadcasted_iota(jnp.int32, (W, 1), 0)
    cols = lax.broadcasted_iota(jnp.int32, (1, H), 1)
    mask = jnp.zeros((W, H), dtype=jnp.bool_)
    for j in range(NB):
        x1 = boxes_ref[i, j, 0] // 8
        y1 = boxes_ref[i, j, 1] // 8
        x2 = boxes_ref[i, j, 2] // 8
        y2 = boxes_ref[i, j, 3] // 8
        rin = (rows >= y1) & (rows < y2)
        cin = (cols >= x1) & (cols < x2)
        mask = mask | (rin & cin)

    lab = jnp.where(mask, jnp.float32(0.0), x)
    bits = lax.bitcast_convert_type(lab, jnp.int32)
    key = jnp.where(bits >= 0, bits, bits ^ _I32_MAXMAG)
    key_ref[0] = jnp.where(bits == _I32_MIN, jnp.int32(0), key)


def _select_kernel(keys_ref, boxes_ref, kstar_ref, idxstar_ref, arr_scr,
                   *, B, W, H, NB):
    keys = keys_ref[...]

    b = boxes_ref[...] // 8
    wdt = b[:, :, 2] - b[:, :, 0]
    hgt = b[:, :, 3] - b[:, :, 1]
    K = jnp.sum(wdt * hgt, axis=1, keepdims=True)[..., None]

    def _count_ge(cand):
        return jnp.sum((keys >= cand).astype(jnp.int32), axis=(1, 2),
                       keepdims=True)

    zero = jnp.zeros((B, 1, 1), jnp.int32)
    T0 = jnp.where(_count_ge(zero) >= K, zero, jnp.full((B, 1, 1), _I32_MIN))

    def _key_body(it, T):
        bit = lax.shift_left(jnp.int32(1), 30 - it)
        cand = T | bit
        return jnp.where(_count_ge(cand) >= K, cand, T)

    kstar = lax.fori_loop(0, 31, _key_body, T0)

    c_gt = jnp.sum((keys > kstar).astype(jnp.int32), axis=(1, 2),
                   keepdims=True)
    c_ge = jnp.sum((keys >= kstar).astype(jnp.int32), axis=(1, 2),
                   keepdims=True)
    t = K - c_gt

    kstar_ref[...] = kstar

    straddle = jnp.sum((c_ge > K).astype(jnp.int32)) > 0

    @pl.when(jnp.logical_not(straddle))
    def _():
        idxstar_ref[...] = jnp.full((B, 1, 1), W * H, jnp.int32)

    @pl.when(straddle)
    def _():
        rows = lax.broadcasted_iota(jnp.int32, (1, W, H), 1)
        cols = lax.broadcasted_iota(jnp.int32, (1, W, H), 2)
        idx = rows * H + cols
        arr_scr[...] = jnp.where(keys == kstar, idx, _I32_MAXMAG)

        def _idx_body(it, p):
            bit = lax.shift_left(jnp.int32(1), 17 - it)
            cand = p | bit
            c = jnp.sum((arr_scr[...] < cand).astype(jnp.int32), axis=(1, 2),
                        keepdims=True)
            return jnp.where(c < t, cand, p)

        idxstar_ref[...] = lax.fori_loop(0, 18, _idx_body,
                                         jnp.zeros((B, 1, 1), jnp.int32))


def _final_kernel(kstar_ref, idxstar_ref, x_ref, key_ref, loss_ref, label_ref,
                  *, W, H):
    i = pl.program_id(0)
    x = x_ref[0]
    key = key_ref[0]
    ks = kstar_ref[i]
    istar = idxstar_ref[i]

    rows = lax.broadcasted_iota(jnp.int32, (W, 1), 0)
    cols = lax.broadcasted_iota(jnp.int32, (1, H), 1)
    idx = rows * H + cols

    sel = (key > ks) | ((key == ks) & (idx <= istar))
    lab = lax.bitcast_convert_type(
        jnp.where(key >= 0, key, key ^ _I32_MAXMAG), jnp.float32)
    label = jnp.where(sel, jnp.float32(1.0), lab)
    label_ref[0] = label

    @pl.when(i == 0)
    def _():
        loss_ref[...] = jnp.zeros((1, 1), jnp.float32)

    loss_ref[...] += jnp.sum(jnp.abs(x - label), keepdims=True)


def kernel(x, boxes):
    B, W, H = x.shape
    NB = boxes.shape[1]
    N = W * H

    keys = pl.pallas_call(
        functools.partial(_keys_kernel, W=W, H=H, NB=NB),
        grid_spec=pltpu.PrefetchScalarGridSpec(
            num_scalar_prefetch=1,
            grid=(B,),
            in_specs=[pl.BlockSpec((1, W, H), lambda i, b: (i, 0, 0))],
            out_specs=pl.BlockSpec((1, W, H), lambda i, b: (i, 0, 0)),
        ),
        out_shape=jax.ShapeDtypeStruct((B, W, H), jnp.int32),
    )(boxes, x)

    kstar, idxstar = pl.pallas_call(
        functools.partial(_select_kernel, B=B, W=W, H=H, NB=NB),
        grid=(1,),
        in_specs=[
            pl.BlockSpec((B, W, H), lambda i: (0, 0, 0)),
            pl.BlockSpec((B, NB, 4), lambda i: (0, 0, 0)),
        ],
        out_specs=[
            pl.BlockSpec((B, 1, 1), lambda i: (0, 0, 0)),
            pl.BlockSpec((B, 1, 1), lambda i: (0, 0, 0)),
        ],
        scratch_shapes=[pltpu.VMEM((B, W, H), jnp.int32)],
        out_shape=[
            jax.ShapeDtypeStruct((B, 1, 1), jnp.int32),
            jax.ShapeDtypeStruct((B, 1, 1), jnp.int32),
        ],
    )(keys, boxes)

    loss_sum, label = pl.pallas_call(
        functools.partial(_final_kernel, W=W, H=H),
        grid_spec=pltpu.PrefetchScalarGridSpec(
            num_scalar_prefetch=2,
            grid=(B,),
            in_specs=[
                pl.BlockSpec((1, W, H), lambda i, a, c: (i, 0, 0)),
                pl.BlockSpec((1, W, H), lambda i, a, c: (i, 0, 0)),
            ],
            out_specs=[
                pl.BlockSpec((1, 1), lambda i, a, c: (0, 0)),
                pl.BlockSpec((1, W, H), lambda i, a, c: (i, 0, 0)),
            ],
        ),
        out_shape=[
            jax.ShapeDtypeStruct((1, 1), jnp.float32),
            jax.ShapeDtypeStruct((B, W, H), jnp.float32),
        ],
    )(kstar.reshape(B), idxstar.reshape(B), x, keys)

    loss = loss_sum[0, 0] / jnp.float32(B * W * H)
    return (loss, x, label)
